# Initial kernel scaffold; baseline (speedup 1.0000x reference)
import jax
import jax.numpy as jnp
from jax.experimental import pallas as pl


def kernel(x_1, edge_index_1, x_1_batch, x_2, edge_index_2, x_2_batch, target, params):
    raise NotImplementedError("write your pallas kernel here")


# speedup vs baseline: 3.8907x; 3.8907x over previous
import functools

import jax
import jax.numpy as jnp
from jax import lax
from jax.experimental import pallas as pl
from jax.experimental.pallas import tpu as pltpu
from jax.experimental.pallas import tpu_sc as plsc

NN = 50000
EE = 800000
BATCH = 128
CHUNK = 4096
NCHUNK = 14
NPAD = CHUNK * NCHUNK
NCORES = 2
NSUB = 16
GRP = 128
EBLK = 2000
EPT = EE // NSUB
NBLK = EPT // EBLK
DEG_ROWS = 6400
NEG = -3.0e38

_sc_mesh = lambda: plsc.VectorSubcoreMesh(
    core_axis_name="c", subcore_axis_name="s",
    num_cores=NCORES, num_subcores=NSUB)


def _sc_degree(col2d):
  slice_w = NPAD // NSUB
  rows_per_w = DEG_ROWS // (NCORES * NSUB)
  n_iters = rows_per_w // 8

  @functools.partial(
      pl.kernel,
      out_type=jax.ShapeDtypeStruct((NCORES, NPAD), jnp.float32),
      mesh=_sc_mesh(),
      compiler_params=pltpu.CompilerParams(needs_layout_passes=False),
      scratch_types=[
          pltpu.VMEM((8, GRP), jnp.int32),
          pltpu.VMEM((GRP,), jnp.float32),
          pltpu.VMEM((slice_w,), jnp.float32),
          pltpu.VMEM_SHARED((NPAD,), jnp.float32),
      ],
  )
  def k(col_ref, out_ref, idx_v, ones_v, stage_v, hist_s):
    core = lax.axis_index("c")
    sid = lax.axis_index("s")
    wid = sid * NCORES + core
    one = jnp.full((16,), 1.0, jnp.float32)
    zero = jnp.zeros((16,), jnp.float32)
    for i in range(GRP // 16):
      ones_v[pl.ds(i * 16, 16)] = one

    def zbody(j, c):
      stage_v[pl.ds(j * 16, 16)] = zero
      return c
    lax.fori_loop(0, slice_w // 16, zbody, 0)
    pltpu.sync_copy(stage_v, hist_s.at[pl.ds(sid * slice_w, slice_w)])
    plsc.subcore_barrier()

    r0 = wid * rows_per_w

    def body(i, c):
      pltpu.sync_copy(col_ref.at[pl.ds(r0 + i * 8, 8)], idx_v)
      for s in range(8):
        pltpu.sync_copy(ones_v, hist_s.at[idx_v.at[s]], add=True)
      return c
    lax.fori_loop(0, n_iters, body, 0)
    plsc.subcore_barrier()
    pltpu.sync_copy(hist_s.at[pl.ds(sid * slice_w, slice_w)],
                    out_ref.at[core, pl.ds(sid * slice_w, slice_w)])

  return k(col2d)


def _sc_aggregate(fp):
  nsr = fp // 128
  ne = GRP // nsr
  pend_n = ((ne + 32 + 15) // 16) * 16
  rpt = CHUNK * nsr // NSUB

  @functools.partial(
      pl.kernel,
      out_type=jax.ShapeDtypeStruct((NPAD * nsr, 128), jnp.float32),
      mesh=_sc_mesh(),
      compiler_params=pltpu.CompilerParams(needs_layout_passes=False),
      scratch_types=[
          pltpu.VMEM((EBLK,), jnp.int32),
          pltpu.VMEM((EBLK,), jnp.int32),
          pltpu.VMEM((pend_n,), jnp.int32),
          pltpu.VMEM((GRP,), jnp.int32),
          pltpu.VMEM((1, GRP), jnp.int32),
          pltpu.VMEM((GRP, 128), jnp.float32),
          pltpu.VMEM_SHARED((CHUNK * nsr + 8, 128), jnp.float32),
          pltpu.SemaphoreType.DMA,
      ],
  )
  def k(y_ref, row_ref, col_ref, out_ref,
        colv, rowv, pend, frow, fcol, rows_v, chunk_s, sem):
    core = lax.axis_index("c")
    sid = lax.axis_index("s")
    ebase = sid * EPT
    iota16 = lax.broadcasted_iota(jnp.int32, (16,), 0)

    def fire():
      for i in range(GRP // 16):
        sub = iota16 + i * 16
        e = sub // nsr
        kk = sub - e * nsr
        pv = plsc.load_gather(pend, [e])
        fr = (pv >> 13) * nsr + kk
        fc = (pv & 8191) * nsr + kk
        if GRP % nsr != 0:
          vok = sub < ne * nsr
          fr = jnp.where(vok, fr, sub - 112)
          fc = jnp.where(vok, fc, CHUNK * nsr + (sub - (ne * nsr)))
        frow[pl.ds(i * 16, 16)] = fr
        fcol[0, pl.ds(i * 16, 16)] = fc
      pltpu.async_copy(y_ref.at[frow], rows_v, sem).wait()
      pltpu.sync_copy(rows_v, chunk_s.at[fcol.at[0]], add=True)

    for ci in range(NCHUNK // NCORES):
      cid = ci * NCORES + core
      lo = cid * CHUNK
      lo_s = cid * CHUNK * nsr
      pltpu.sync_copy(y_ref.at[pl.ds(lo_s + sid * rpt, rpt)],
                      chunk_s.at[pl.ds(sid * rpt, rpt)])
      plsc.subcore_barrier()

      def vec_body(j, pos):
        c16 = colv[pl.ds(j * 16, 16)]
        r16 = rowv[pl.ds(j * 16, 16)]
        m = (c16 >= lo) & (c16 < lo + CHUNK)
        val = r16 * 8192 + (c16 - lo)
        pfx = plsc.cumsum(m.astype(jnp.int32))
        dst = pos + pfx - 1
        plsc.store_scatter(pend, [dst], val, mask=m)
        pos = pos + pfx[15]
        full = pos >= ne

        @pl.when(full)
        def _():
          fire()
          pend[pl.ds(0, 16)] = pend[pl.ds(ne, 16)]

        return jnp.where(full, pos - ne, pos)

      def blk_body(bi, pos):
        eb = ebase + bi * EBLK
        pltpu.sync_copy(col_ref.at[pl.ds(eb, EBLK)], colv)
        pltpu.sync_copy(row_ref.at[pl.ds(eb, EBLK)], rowv)
        return lax.fori_loop(0, EBLK // 16, vec_body, pos)

      pos = lax.fori_loop(0, NBLK, blk_body, jnp.int32(0))

      for i in range(-(-ne // 16)):
        off = i * 16
        lanes = iota16 + off
        sel = lanes < pos
        cc = pend[pl.ds(off, 16)]
        pend[pl.ds(off, 16)] = jnp.where(sel, cc, lanes * 8192 + CHUNK)
      fire()
      plsc.subcore_barrier()
      pltpu.sync_copy(chunk_s.at[pl.ds(sid * rpt, rpt)],
                      out_ref.at[pl.ds(lo_s + sid * rpt, rpt)])

  return k


def _tc_dinv(parts):
  p3 = parts.reshape(2, NPAD // 128, 128)

  def body(p_ref, o_ref):
    deg = 1.0 + p_ref[0] + p_ref[1]
    o_ref[...] = lax.rsqrt(deg)

  out = pl.pallas_call(
      body,
      out_shape=jax.ShapeDtypeStruct((NPAD // 128, 128), jnp.float32),
  )(p3)
  return out.reshape(NPAD, 1)


_MBLK = 512


def _tc_mm_scale(x, w, dinv):
  fi, fo = w.shape

  def body(x_ref, d_ref, w_ref, o_ref):
    o_ref[...] = jnp.dot(x_ref[...], w_ref[...],
                         preferred_element_type=jnp.float32) * d_ref[...]

  return pl.pallas_call(
      body,
      grid=(NPAD // _MBLK,),
      in_specs=[
          pl.BlockSpec((_MBLK, fi), lambda i: (i, 0)),
          pl.BlockSpec((_MBLK, 1), lambda i: (i, 0)),
          pl.BlockSpec((fi, fo), lambda i: (0, 0)),
      ],
      out_specs=pl.BlockSpec((_MBLK, fo), lambda i: (i, 0)),
      out_shape=jax.ShapeDtypeStruct((NPAD, fo), jnp.float32),
  )(x, dinv, w)


def _tc_gcn_mm(agg, dinv, b8, w):
  fi, fo = w.shape

  def body(a_ref, d_ref, b_ref, w_ref, o_ref):
    d = d_ref[...]
    h = jnp.maximum(a_ref[...] * d + b_ref[0:1, :], 0.0)
    o_ref[...] = jnp.dot(h, w_ref[...],
                         preferred_element_type=jnp.float32) * d

  return pl.pallas_call(
      body,
      grid=(NPAD // _MBLK,),
      in_specs=[
          pl.BlockSpec((_MBLK, fi), lambda i: (i, 0)),
          pl.BlockSpec((_MBLK, 1), lambda i: (i, 0)),
          pl.BlockSpec((8, fi), lambda i: (0, 0)),
          pl.BlockSpec((fi, fo), lambda i: (0, 0)),
      ],
      out_specs=pl.BlockSpec((_MBLK, fo), lambda i: (i, 0)),
      out_shape=jax.ShapeDtypeStruct((NPAD, fo), jnp.float32),
  )(agg, dinv, b8, w)


def _tc_segmax(agg, dinv, b8, bv):
  fp = agg.shape[1]

  def body(a_ref, d_ref, b_ref, bv_ref, o_ref):
    i = pl.program_id(0)

    @pl.when(i == 0)
    def _():
      o_ref[...] = jnp.full((BATCH, fp), NEG, jnp.float32)

    h = jnp.maximum(a_ref[...] * d_ref[...] + b_ref[0:1, :], 0.0)
    rows = i * _MBLK + lax.broadcasted_iota(jnp.int32, (_MBLK, 1), 0)
    h = jnp.where(rows < NN, h, NEG)
    bvb = bv_ref[...]
    bmin = bv_ref[0, 0]
    bmax = bv_ref[_MBLK - 1, 0]

    def b_body(bb, c):
      hm = jnp.where(bvb == bb, h, NEG)
      red = jnp.max(hm, axis=0, keepdims=True)
      o_ref[pl.ds(bb, 1), :] = jnp.maximum(o_ref[pl.ds(bb, 1), :], red)
      return c

    lax.fori_loop(bmin, bmax + 1, b_body, 0)

  return pl.pallas_call(
      body,
      grid=(NPAD // _MBLK,),
      in_specs=[
          pl.BlockSpec((_MBLK, fp), lambda i: (i, 0)),
          pl.BlockSpec((_MBLK, 1), lambda i: (i, 0)),
          pl.BlockSpec((8, fp), lambda i: (0, 0)),
          pl.BlockSpec((_MBLK, 1), lambda i: (i, 0)),
      ],
      out_specs=pl.BlockSpec((BATCH, fp), lambda i: (0, 0)),
      out_shape=jax.ShapeDtypeStruct((BATCH, fp), jnp.float32),
  )(agg, dinv, b8, bv)


def _tc_fcg(seg, w1, b1, w2, b2):
  def body(s_ref, w1_ref, b1_ref, w2_ref, b2_ref, o_ref):
    h = jnp.maximum(
        jnp.dot(s_ref[...], w1_ref[...], preferred_element_type=jnp.float32)
        + b1_ref[0:1, :], 0.0)
    o_ref[...] = jnp.dot(h, w2_ref[...],
                         preferred_element_type=jnp.float32) + b2_ref[0:1, :]

  return pl.pallas_call(
      body,
      out_shape=jax.ShapeDtypeStruct((BATCH, w2.shape[1]), jnp.float32),
  )(seg, w1, b1, w2, b2)


_TB = 128


def _tc_conv_halo(xs, wT, b128, k, stride, tout):
  co = wT.shape[1]
  grid = -(-tout // _TB)
  halo = (k - 1) if stride == 1 else (k - 1) // 2
  hl = halo * BATCH
  wblk = _TB * BATCH

  def body(*refs):
    n = len(xs)
    x_refs = refs[:2 * n]
    wT_ref, b_ref, o_ref = refs[2 * n], refs[2 * n + 1], refs[2 * n + 2]
    cats = []
    for t in range(n):
      cur = x_refs[2 * t][...]
      nxt = x_refs[2 * t + 1][:, 0:hl]
      cats.append(jnp.concatenate([cur, nxt], axis=1))
    y = None
    for j in range(k):
      if stride == 1:
        sl = cats[0][:, j * BATCH:j * BATCH + wblk]
      else:
        sl = cats[j % 2][:, (j // 2) * BATCH:(j // 2) * BATCH + wblk]
      t = jnp.dot(wT_ref[j], sl, preferred_element_type=jnp.float32)
      y = t if y is None else y + t
    o_ref[...] = y + b_ref[:, 0:1]

  in_specs = []
  for x in xs:
    ci = x.shape[0]
    in_specs.append(pl.BlockSpec((ci, wblk), lambda i: (0, i)))
    in_specs.append(pl.BlockSpec((ci, wblk), lambda i: (0, i + 1)))
  in_specs.append(pl.BlockSpec(wT.shape, lambda i: (0, 0, 0)))
  in_specs.append(pl.BlockSpec(b128.shape, lambda i: (0, 0)))

  args = []
  for x in xs:
    args += [x, x]
  return pl.pallas_call(
      body,
      grid=(grid,),
      in_specs=in_specs,
      out_specs=pl.BlockSpec((co, wblk), lambda i: (0, i)),
      out_shape=jax.ShapeDtypeStruct((co, tout * BATCH), jnp.float32),
  )(*args, wT, b128)


def _tc_conv_whole(x, wT, b128, k, tout):
  co = wT.shape[1]
  wout = tout * BATCH

  def body(x_ref, wT_ref, b_ref, o_ref):
    xx = x_ref[...]
    y = None
    for j in range(k):
      sl = xx[:, j * BATCH:j * BATCH + wout]
      t = jnp.dot(wT_ref[j], sl, preferred_element_type=jnp.float32)
      y = t if y is None else y + t
    o_ref[...] = y + b_ref[:, 0:1]

  return pl.pallas_call(
      body,
      out_shape=jax.ShapeDtypeStruct((co, wout), jnp.float32),
  )(x, wT, b128)


def _tc_pool(phases):
  co, w = phases[0].shape
  wblk = min(w, 16 * 1024)
  grid = -(-w // wblk)

  def body(*refs):
    o_ref = refs[-1]
    m = refs[0][...]
    for r in refs[1:-1]:
      m = jnp.maximum(m, r[...])
    o_ref[...] = m

  return pl.pallas_call(
      body,
      grid=(grid,),
      in_specs=[pl.BlockSpec((co, wblk), lambda i: (0, i))
                for _ in phases],
      out_specs=pl.BlockSpec((co, wblk), lambda i: (0, i)),
      out_shape=jax.ShapeDtypeStruct((co, w), jnp.float32),
  )(*phases)


_CNN_CFG = [
    (8, 2, 3, 32, 8497, 2832),
    (8, 2, 3, 32, 1413, 471),
    (4, 1, 3, 64, 468, 156),
    (4, 1, 2, 64, 153, 76),
    (4, 1, 2, 128, 73, 36),
    (2, 1, 2, 128, 35, 17),
    (2, 1, 2, 64, 16, 8),
]


def _cnn_branch(target, p):
  x3 = target.T[None]
  for li, (k, s, pool, co, tout, tpool) in enumerate(_CNN_CFG):
    wT = jnp.transpose(p[f"c{li + 1}_W"], (2, 0, 1))
    b128 = jnp.tile(p[f"c{li + 1}_b"][:, None], (1, BATCH))
    ci = x3.shape[0]
    if s == 2:
      grid = -(-tout // _TB)
      tpad = (grid + 1) * _TB
      xe = x3[:, 0::2, :]
      xo = x3[:, 1::2, :]
      xe = jnp.pad(xe, ((0, 0), (0, tpad - xe.shape[1]), (0, 0)))
      xo = jnp.pad(xo, ((0, 0), (0, tpad - xo.shape[1]), (0, 0)))
      y = _tc_conv_halo([xe.reshape(ci, -1), xo.reshape(ci, -1)],
                        wT, b128, k, 2, tout)
    else:
      y = _tc_conv_whole(x3.reshape(ci, -1), wT, b128, k, tout)
    y3 = y.reshape(co, tout, BATCH)
    phases = [y3[:, j:tpool * pool:pool, :].reshape(co, -1)
              for j in range(pool)]
    yp = _tc_pool(phases)
    x3 = yp.reshape(co, tpool, BATCH)
  return x3.reshape(512, BATCH).T


def _tc_head(g1, g2, flatT, wx, bx, w1a, w1b, w1c, b1, w2, b2, w3, b3):
  def body(g1_ref, g2_ref, f_ref, wx_ref, bx_ref, w1a_ref, w1b_ref, w1c_ref,
           b1_ref, w2_ref, b2_ref, w3_ref, b3_ref, o_ref):
    f32 = jnp.float32
    xt = jnp.dot(f_ref[...], wx_ref[...],
                 preferred_element_type=f32) + bx_ref[0:1, :]
    h = (jnp.dot(g1_ref[...], w1a_ref[...], preferred_element_type=f32)
         + jnp.dot(g2_ref[...], w1b_ref[...], preferred_element_type=f32)
         + jnp.dot(xt, w1c_ref[...], preferred_element_type=f32)
         + b1_ref[0:1, :])
    h = jnp.maximum(h, 0.0)
    h = jnp.maximum(
        jnp.dot(h, w2_ref[...], preferred_element_type=f32) + b2_ref[0:1, :],
        0.0)
    o_ref[...] = jnp.dot(h, w3_ref[...],
                         preferred_element_type=f32) + b3_ref[0:1, :]

  return pl.pallas_call(
      body,
      out_shape=jax.ShapeDtypeStruct((BATCH, 1), jnp.float32),
  )(g1, g2, flatT, wx, bx, w1a, w1b, w1c, b1, w2, b2, w3, b3)


def _pad_rows_cols(w, fi, fo):
  return jnp.pad(w, ((0, fi - w.shape[0]), (0, fo - w.shape[1])))


def _bias8(b, fo):
  return jnp.tile(jnp.pad(b, (0, fo - b.shape[0]))[None, :], (8, 1))


def _branch(x, ei, bv, p, fps):
  f1, f2, f3 = fps
  row = ei[0]
  col = ei[1]
  col_pad = jnp.concatenate(
      [col, NN + (jnp.arange(DEG_ROWS * GRP - EE, dtype=jnp.int32) % 7000)])
  parts = _sc_degree(col_pad.reshape(DEG_ROWS, GRP))
  dinv = _tc_dinv(parts)

  def agg_call(y, fp):
    y2 = y.reshape(NPAD * (fp // 128), 128)
    return _sc_aggregate(fp)(y2, row, col).reshape(NPAD, fp)

  x_pad = jnp.pad(x, ((0, NPAD - NN), (0, f1 - x.shape[1])))
  w1 = _pad_rows_cols(p["gcn1_W"], f1, f1)
  y1 = _tc_mm_scale(x_pad, w1, dinv)
  agg1 = agg_call(y1, f1)

  w2 = _pad_rows_cols(p["gcn2_W"], f1, f2)
  y2 = _tc_gcn_mm(agg1, dinv, _bias8(p["gcn1_b"], f1), w2)
  agg2 = agg_call(y2, f2)

  w3 = _pad_rows_cols(p["gcn3_W"], f2, f3)
  y3 = _tc_gcn_mm(agg2, dinv, _bias8(p["gcn2_b"], f2), w3)
  agg3 = agg_call(y3, f3)

  bv_pad = jnp.pad(bv, (0, NPAD - NN), constant_values=BATCH - 1)
  seg = _tc_segmax(agg3, dinv, _bias8(p["gcn3_b"], f3), bv_pad[:, None])

  w_f1 = jnp.pad(p["fcg1_W"], ((0, f3 - p["fcg1_W"].shape[0]), (0, 0)))
  return _tc_fcg(seg, w_f1, _bias8(p["fcg1_b"], 1024),
                 p["fcg2_W"], _bias8(p["fcg2_b"], 128))


def kernel(x_1, edge_index_1, x_1_batch, x_2, edge_index_2, x_2_batch,
           target, params):
  p = params
  fps = (128, 256, 384)
  g1 = _branch(x_1, edge_index_1, x_1_batch, p, fps)
  g2 = _branch(x_2, edge_index_2, x_2_batch, p, fps)

  flatT = _cnn_branch(target, p)

  w1 = p["fc1_W"]
  out = _tc_head(g1, g2, flatT, p["fcxt_W"], _bias8(p["fcxt_b"], 128),
                 w1[0:128], w1[128:256], w1[256:384],
                 _bias8(p["fc1_b"], 1024), p["fc2_W"],
                 _bias8(p["fc2_b"], 128), p["out_W"], _bias8(p["out_b"], 1))
  return out


# speedup vs baseline: 4.3590x; 1.1204x over previous
import functools

import jax
import jax.numpy as jnp
from jax import lax
from jax.experimental import pallas as pl
from jax.experimental.pallas import tpu as pltpu
from jax.experimental.pallas import tpu_sc as plsc

NN = 50000
EE = 800000
BATCH = 128
NPAD = 57344
NCORES = 2
NSUB = 16
GRP = 128
EBLK = 10000
EPT = EE // NSUB
NBLK = EPT // EBLK
DEG_ROWS = 6400
NEG = -3.0e38

_sc_mesh = lambda: plsc.VectorSubcoreMesh(
    core_axis_name="c", subcore_axis_name="s",
    num_cores=NCORES, num_subcores=NSUB)


def _sc_degree(col2d):
  slice_w = NPAD // NSUB
  rows_per_w = DEG_ROWS // (NCORES * NSUB)
  n_iters = rows_per_w // 40

  @functools.partial(
      pl.kernel,
      out_type=jax.ShapeDtypeStruct((NCORES, NPAD), jnp.float32),
      mesh=_sc_mesh(),
      compiler_params=pltpu.CompilerParams(needs_layout_passes=False),
      scratch_types=[
          pltpu.VMEM((40, GRP), jnp.int32),
          pltpu.VMEM((GRP,), jnp.float32),
          pltpu.VMEM((slice_w,), jnp.float32),
          pltpu.VMEM_SHARED((NPAD,), jnp.float32),
          pltpu.SemaphoreType.DMA,
      ],
  )
  def k(col_ref, out_ref, idx_v, ones_v, stage_v, hist_s, sem):
    core = lax.axis_index("c")
    sid = lax.axis_index("s")
    wid = sid * NCORES + core
    one = jnp.full((16,), 1.0, jnp.float32)
    zero = jnp.zeros((16,), jnp.float32)
    for i in range(GRP // 16):
      ones_v[pl.ds(i * 16, 16)] = one

    def zbody(j, c):
      stage_v[pl.ds(j * 16, 16)] = zero
      return c
    lax.fori_loop(0, slice_w // 16, zbody, 0)
    pltpu.sync_copy(stage_v, hist_s.at[pl.ds(sid * slice_w, slice_w)])
    plsc.subcore_barrier()

    r0 = wid * rows_per_w

    def body(i, c):
      pltpu.sync_copy(col_ref.at[pl.ds(r0 + i * 40, 40)], idx_v)
      descs = [pltpu.async_copy(ones_v, hist_s.at[idx_v.at[s]], sem, add=True)
               for s in range(40)]
      for d in descs:
        d.wait()
      return c
    lax.fori_loop(0, n_iters, body, 0)
    plsc.subcore_barrier()
    pltpu.sync_copy(hist_s.at[pl.ds(sid * slice_w, slice_w)],
                    out_ref.at[core, pl.ds(sid * slice_w, slice_w)])

  return k(col2d)


def _sc_aggregate(fp):
  nsr = fp // 128
  chunk = {128: 7168, 256: 3584, 384: 3584}[fp]
  nchunk = NPAD // chunk
  ne = GRP // nsr
  pend_n = ((ne + 32 + 15) // 16) * 16
  rpt = chunk * nsr // NSUB

  @functools.partial(
      pl.kernel,
      out_type=jax.ShapeDtypeStruct((NPAD * nsr, 128), jnp.float32),
      mesh=_sc_mesh(),
      compiler_params=pltpu.CompilerParams(needs_layout_passes=False),
      scratch_types=[
          pltpu.VMEM((EBLK,), jnp.int32),
          pltpu.VMEM((EBLK,), jnp.int32),
          pltpu.VMEM((pend_n,), jnp.int32),
          pltpu.VMEM((GRP,), jnp.int32),
          pltpu.VMEM((1, GRP), jnp.int32),
          pltpu.VMEM((GRP, 128), jnp.float32),
          pltpu.VMEM_SHARED((chunk * nsr + 8, 128), jnp.float32),
          pltpu.SemaphoreType.DMA,
      ],
  )
  def k(y_ref, row_ref, col_ref, out_ref,
        colv, rowv, pend, frow, fcol, rows_v, chunk_s, sem):
    core = lax.axis_index("c")
    sid = lax.axis_index("s")
    ebase = sid * EPT
    iota16 = lax.broadcasted_iota(jnp.int32, (16,), 0)

    def fire():
      for i in range(GRP // 16):
        sub = iota16 + i * 16
        e = sub // nsr
        kk = sub - e * nsr
        pv = plsc.load_gather(pend, [e])
        fr = (pv >> 14) * nsr + kk
        fc = (pv & 16383) * nsr + kk
        if GRP % nsr != 0:
          vok = sub < ne * nsr
          fr = jnp.where(vok, fr, sub - 112)
          fc = jnp.where(vok, fc, chunk * nsr + (sub - (ne * nsr)))
        frow[pl.ds(i * 16, 16)] = fr
        fcol[0, pl.ds(i * 16, 16)] = fc
      pltpu.async_copy(y_ref.at[frow], rows_v, sem).wait()
      pltpu.sync_copy(rows_v, chunk_s.at[fcol.at[0]], add=True)

    for ci in range(nchunk // NCORES):
      cid = ci * NCORES + core
      lo = cid * chunk
      lo_s = cid * chunk * nsr
      pltpu.sync_copy(y_ref.at[pl.ds(lo_s + sid * rpt, rpt)],
                      chunk_s.at[pl.ds(sid * rpt, rpt)])
      plsc.subcore_barrier()

      def vec_body(j, pos):
        c16 = colv[pl.ds(j * 16, 16)]
        r16 = rowv[pl.ds(j * 16, 16)]
        m = (c16 >= lo) & (c16 < lo + chunk)
        val = r16 * 16384 + (c16 - lo)
        pfx = plsc.cumsum(m.astype(jnp.int32))
        dst = pos + pfx - 1
        plsc.store_scatter(pend, [dst], val, mask=m)
        pos = pos + pfx[15]
        full = pos >= ne

        @pl.when(full)
        def _():
          fire()
          pend[pl.ds(0, 16)] = pend[pl.ds(ne, 16)]

        return jnp.where(full, pos - ne, pos)

      def blk_body(bi, pos):
        eb = ebase + bi * EBLK
        pltpu.sync_copy(col_ref.at[pl.ds(eb, EBLK)], colv)
        pltpu.sync_copy(row_ref.at[pl.ds(eb, EBLK)], rowv)
        return lax.fori_loop(0, EBLK // 16, vec_body, pos)

      pos = lax.fori_loop(0, NBLK, blk_body, jnp.int32(0))

      for i in range(-(-ne // 16)):
        off = i * 16
        lanes = iota16 + off
        sel = lanes < pos
        cc = pend[pl.ds(off, 16)]
        pend[pl.ds(off, 16)] = jnp.where(sel, cc, lanes * 16384 + chunk)
      fire()
      plsc.subcore_barrier()
      pltpu.sync_copy(chunk_s.at[pl.ds(sid * rpt, rpt)],
                      out_ref.at[pl.ds(lo_s + sid * rpt, rpt)])

  return k


def _tc_dinv(parts):
  p3 = parts.reshape(2, NPAD // 128, 128)

  def body(p_ref, o_ref):
    deg = 1.0 + p_ref[0] + p_ref[1]
    o_ref[...] = lax.rsqrt(deg)

  out = pl.pallas_call(
      body,
      out_shape=jax.ShapeDtypeStruct((NPAD // 128, 128), jnp.float32),
  )(p3)
  return out.reshape(NPAD, 1)


_MBLK = 512


def _tc_mm_scale(x, w, dinv):
  fi, fo = w.shape

  def body(x_ref, d_ref, w_ref, o_ref):
    o_ref[...] = jnp.dot(x_ref[...], w_ref[...],
                         preferred_element_type=jnp.float32) * d_ref[...]

  return pl.pallas_call(
      body,
      grid=(NPAD // _MBLK,),
      in_specs=[
          pl.BlockSpec((_MBLK, fi), lambda i: (i, 0)),
          pl.BlockSpec((_MBLK, 1), lambda i: (i, 0)),
          pl.BlockSpec((fi, fo), lambda i: (0, 0)),
      ],
      out_specs=pl.BlockSpec((_MBLK, fo), lambda i: (i, 0)),
      out_shape=jax.ShapeDtypeStruct((NPAD, fo), jnp.float32),
  )(x, dinv, w)


def _tc_gcn_mm(agg, dinv, b8, w):
  fi, fo = w.shape

  def body(a_ref, d_ref, b_ref, w_ref, o_ref):
    d = d_ref[...]
    h = jnp.maximum(a_ref[...] * d + b_ref[0:1, :], 0.0)
    o_ref[...] = jnp.dot(h, w_ref[...],
                         preferred_element_type=jnp.float32) * d

  return pl.pallas_call(
      body,
      grid=(NPAD // _MBLK,),
      in_specs=[
          pl.BlockSpec((_MBLK, fi), lambda i: (i, 0)),
          pl.BlockSpec((_MBLK, 1), lambda i: (i, 0)),
          pl.BlockSpec((8, fi), lambda i: (0, 0)),
          pl.BlockSpec((fi, fo), lambda i: (0, 0)),
      ],
      out_specs=pl.BlockSpec((_MBLK, fo), lambda i: (i, 0)),
      out_shape=jax.ShapeDtypeStruct((NPAD, fo), jnp.float32),
  )(agg, dinv, b8, w)


def _tc_segmax(agg, dinv, b8, bv):
  fp = agg.shape[1]

  def body(a_ref, d_ref, b_ref, bv_ref, o_ref):
    i = pl.program_id(0)

    @pl.when(i == 0)
    def _():
      o_ref[...] = jnp.full((BATCH, fp), NEG, jnp.float32)

    h = jnp.maximum(a_ref[...] * d_ref[...] + b_ref[0:1, :], 0.0)
    rows = i * _MBLK + lax.broadcasted_iota(jnp.int32, (_MBLK, 1), 0)
    h = jnp.where(rows < NN, h, NEG)
    bvb = bv_ref[...]
    bmin = bv_ref[0, 0]
    bmax = bv_ref[_MBLK - 1, 0]

    def b_body(bb, c):
      hm = jnp.where(bvb == bb, h, NEG)
      red = jnp.max(hm, axis=0, keepdims=True)
      o_ref[pl.ds(bb, 1), :] = jnp.maximum(o_ref[pl.ds(bb, 1), :], red)
      return c

    lax.fori_loop(bmin, bmax + 1, b_body, 0)

  return pl.pallas_call(
      body,
      grid=(NPAD // _MBLK,),
      in_specs=[
          pl.BlockSpec((_MBLK, fp), lambda i: (i, 0)),
          pl.BlockSpec((_MBLK, 1), lambda i: (i, 0)),
          pl.BlockSpec((8, fp), lambda i: (0, 0)),
          pl.BlockSpec((_MBLK, 1), lambda i: (i, 0)),
      ],
      out_specs=pl.BlockSpec((BATCH, fp), lambda i: (0, 0)),
      out_shape=jax.ShapeDtypeStruct((BATCH, fp), jnp.float32),
  )(agg, dinv, b8, bv)


def _tc_fcg(seg, w1, b1, w2, b2):
  def body(s_ref, w1_ref, b1_ref, w2_ref, b2_ref, o_ref):
    h = jnp.maximum(
        jnp.dot(s_ref[...], w1_ref[...], preferred_element_type=jnp.float32)
        + b1_ref[0:1, :], 0.0)
    o_ref[...] = jnp.dot(h, w2_ref[...],
                         preferred_element_type=jnp.float32) + b2_ref[0:1, :]

  return pl.pallas_call(
      body,
      out_shape=jax.ShapeDtypeStruct((BATCH, w2.shape[1]), jnp.float32),
  )(seg, w1, b1, w2, b2)


_TB = 128


def _tc_conv_halo(xs, wT, b128, k, stride, tout):
  co = wT.shape[1]
  grid = -(-tout // _TB)
  halo = (k - 1) if stride == 1 else (k - 1) // 2
  hl = halo * BATCH
  wblk = _TB * BATCH

  def body(*refs):
    n = len(xs)
    x_refs = refs[:2 * n]
    wT_ref, b_ref, o_ref = refs[2 * n], refs[2 * n + 1], refs[2 * n + 2]
    cats = []
    for t in range(n):
      cur = x_refs[2 * t][...]
      nxt = x_refs[2 * t + 1][:, 0:hl]
      cats.append(jnp.concatenate([cur, nxt], axis=1))
    y = None
    for j in range(k):
      if stride == 1:
        sl = cats[0][:, j * BATCH:j * BATCH + wblk]
      else:
        sl = cats[j % 2][:, (j // 2) * BATCH:(j // 2) * BATCH + wblk]
      t = jnp.dot(wT_ref[j], sl, preferred_element_type=jnp.float32)
      y = t if y is None else y + t
    o_ref[...] = y + b_ref[:, 0:1]

  in_specs = []
  for x in xs:
    ci = x.shape[0]
    in_specs.append(pl.BlockSpec((ci, wblk), lambda i: (0, i)))
    in_specs.append(pl.BlockSpec((ci, wblk), lambda i: (0, i + 1)))
  in_specs.append(pl.BlockSpec(wT.shape, lambda i: (0, 0, 0)))
  in_specs.append(pl.BlockSpec(b128.shape, lambda i: (0, 0)))

  args = []
  for x in xs:
    args += [x, x]
  return pl.pallas_call(
      body,
      grid=(grid,),
      in_specs=in_specs,
      out_specs=pl.BlockSpec((co, wblk), lambda i: (0, i)),
      out_shape=jax.ShapeDtypeStruct((co, tout * BATCH), jnp.float32),
  )(*args, wT, b128)


def _tc_conv_whole(x, wT, b128, k, tout):
  co = wT.shape[1]
  wout = tout * BATCH

  def body(x_ref, wT_ref, b_ref, o_ref):
    xx = x_ref[...]
    y = None
    for j in range(k):
      sl = xx[:, j * BATCH:j * BATCH + wout]
      t = jnp.dot(wT_ref[j], sl, preferred_element_type=jnp.float32)
      y = t if y is None else y + t
    o_ref[...] = y + b_ref[:, 0:1]

  return pl.pallas_call(
      body,
      out_shape=jax.ShapeDtypeStruct((co, wout), jnp.float32),
  )(x, wT, b128)


def _tc_pool(phases):
  co, w = phases[0].shape
  wblk = min(w, 16 * 1024)
  grid = -(-w // wblk)

  def body(*refs):
    o_ref = refs[-1]
    m = refs[0][...]
    for r in refs[1:-1]:
      m = jnp.maximum(m, r[...])
    o_ref[...] = m

  return pl.pallas_call(
      body,
      grid=(grid,),
      in_specs=[pl.BlockSpec((co, wblk), lambda i: (0, i))
                for _ in phases],
      out_specs=pl.BlockSpec((co, wblk), lambda i: (0, i)),
      out_shape=jax.ShapeDtypeStruct((co, w), jnp.float32),
  )(*phases)


_CNN_CFG = [
    (8, 2, 3, 32, 8497, 2832),
    (8, 2, 3, 32, 1413, 471),
    (4, 1, 3, 64, 468, 156),
    (4, 1, 2, 64, 153, 76),
    (4, 1, 2, 128, 73, 36),
    (2, 1, 2, 128, 35, 17),
    (2, 1, 2, 64, 16, 8),
]


def _cnn_branch(target, p):
  x3 = target.T[None]
  for li, (k, s, pool, co, tout, tpool) in enumerate(_CNN_CFG):
    wT = jnp.transpose(p[f"c{li + 1}_W"], (2, 0, 1))
    b128 = jnp.tile(p[f"c{li + 1}_b"][:, None], (1, BATCH))
    ci = x3.shape[0]
    if s == 2:
      grid = -(-tout // _TB)
      tpad = (grid + 1) * _TB
      xe = x3[:, 0::2, :]
      xo = x3[:, 1::2, :]
      xe = jnp.pad(xe, ((0, 0), (0, tpad - xe.shape[1]), (0, 0)))
      xo = jnp.pad(xo, ((0, 0), (0, tpad - xo.shape[1]), (0, 0)))
      y = _tc_conv_halo([xe.reshape(ci, -1), xo.reshape(ci, -1)],
                        wT, b128, k, 2, tout)
    else:
      y = _tc_conv_whole(x3.reshape(ci, -1), wT, b128, k, tout)
    y3 = y.reshape(co, tout, BATCH)
    phases = [y3[:, j:tpool * pool:pool, :].reshape(co, -1)
              for j in range(pool)]
    yp = _tc_pool(phases)
    x3 = yp.reshape(co, tpool, BATCH)
  return x3.reshape(512, BATCH).T


def _tc_head(g1, g2, flatT, wx, bx, w1a, w1b, w1c, b1, w2, b2, w3, b3):
  def body(g1_ref, g2_ref, f_ref, wx_ref, bx_ref, w1a_ref, w1b_ref, w1c_ref,
           b1_ref, w2_ref, b2_ref, w3_ref, b3_ref, o_ref):
    f32 = jnp.float32
    xt = jnp.dot(f_ref[...], wx_ref[...],
                 preferred_element_type=f32) + bx_ref[0:1, :]
    h = (jnp.dot(g1_ref[...], w1a_ref[...], preferred_element_type=f32)
         + jnp.dot(g2_ref[...], w1b_ref[...], preferred_element_type=f32)
         + jnp.dot(xt, w1c_ref[...], preferred_element_type=f32)
         + b1_ref[0:1, :])
    h = jnp.maximum(h, 0.0)
    h = jnp.maximum(
        jnp.dot(h, w2_ref[...], preferred_element_type=f32) + b2_ref[0:1, :],
        0.0)
    o_ref[...] = jnp.dot(h, w3_ref[...],
                         preferred_element_type=f32) + b3_ref[0:1, :]

  return pl.pallas_call(
      body,
      out_shape=jax.ShapeDtypeStruct((BATCH, 1), jnp.float32),
  )(g1, g2, flatT, wx, bx, w1a, w1b, w1c, b1, w2, b2, w3, b3)


def _pad_rows_cols(w, fi, fo):
  return jnp.pad(w, ((0, fi - w.shape[0]), (0, fo - w.shape[1])))


def _bias8(b, fo):
  return jnp.tile(jnp.pad(b, (0, fo - b.shape[0]))[None, :], (8, 1))


def _branch(x, ei, bv, p, fps):
  f1, f2, f3 = fps
  row = ei[0]
  col = ei[1]
  col_pad = jnp.concatenate(
      [col, NN + (jnp.arange(DEG_ROWS * GRP - EE, dtype=jnp.int32) % 7000)])
  parts = _sc_degree(col_pad.reshape(DEG_ROWS, GRP))
  dinv = _tc_dinv(parts)

  def agg_call(y, fp):
    y2 = y.reshape(NPAD * (fp // 128), 128)
    return _sc_aggregate(fp)(y2, row, col).reshape(NPAD, fp)

  x_pad = jnp.pad(x, ((0, NPAD - NN), (0, f1 - x.shape[1])))
  w1 = _pad_rows_cols(p["gcn1_W"], f1, f1)
  y1 = _tc_mm_scale(x_pad, w1, dinv)
  agg1 = agg_call(y1, f1)

  w2 = _pad_rows_cols(p["gcn2_W"], f1, f2)
  y2 = _tc_gcn_mm(agg1, dinv, _bias8(p["gcn1_b"], f1), w2)
  agg2 = agg_call(y2, f2)

  w3 = _pad_rows_cols(p["gcn3_W"], f2, f3)
  y3 = _tc_gcn_mm(agg2, dinv, _bias8(p["gcn2_b"], f2), w3)
  agg3 = agg_call(y3, f3)

  bv_pad = jnp.pad(bv, (0, NPAD - NN), constant_values=BATCH - 1)
  seg = _tc_segmax(agg3, dinv, _bias8(p["gcn3_b"], f3), bv_pad[:, None])

  w_f1 = jnp.pad(p["fcg1_W"], ((0, f3 - p["fcg1_W"].shape[0]), (0, 0)))
  return _tc_fcg(seg, w_f1, _bias8(p["fcg1_b"], 1024),
                 p["fcg2_W"], _bias8(p["fcg2_b"], 128))


def kernel(x_1, edge_index_1, x_1_batch, x_2, edge_index_2, x_2_batch,
           target, params):
  p = params
  fps = (128, 256, 384)
  g1 = _branch(x_1, edge_index_1, x_1_batch, p, fps)
  g2 = _branch(x_2, edge_index_2, x_2_batch, p, fps)

  flatT = _cnn_branch(target, p)

  w1 = p["fc1_W"]
  out = _tc_head(g1, g2, flatT, p["fcxt_W"], _bias8(p["fcxt_b"], 128),
                 w1[0:128], w1[128:256], w1[256:384],
                 _bias8(p["fc1_b"], 1024), p["fc2_W"],
                 _bias8(p["fc2_b"], 128), p["out_W"], _bias8(p["out_b"], 1))
  return out


# speedup vs baseline: 5.3364x; 1.2242x over previous
import functools

import jax
import jax.numpy as jnp
from jax import lax
from jax.experimental import pallas as pl
from jax.experimental.pallas import tpu as pltpu
from jax.experimental.pallas import tpu_sc as plsc

NN = 50000
EE = 800000
BATCH = 128
NPAD = 57344
NCORES = 2
NSUB = 16
GRP = 128
EBLK = 10000
EPT = EE // NSUB
NBLK = EPT // EBLK
DEG_ROWS = 6400
NEG = -3.0e38

_sc_mesh = lambda: plsc.VectorSubcoreMesh(
    core_axis_name="c", subcore_axis_name="s",
    num_cores=NCORES, num_subcores=NSUB)


def _sc_degree(col2d):
  slice_w = NPAD // NSUB
  rows_per_w = DEG_ROWS // (NCORES * NSUB)
  n_iters = rows_per_w // 40

  @functools.partial(
      pl.kernel,
      out_type=jax.ShapeDtypeStruct((NCORES, NPAD), jnp.float32),
      mesh=_sc_mesh(),
      compiler_params=pltpu.CompilerParams(needs_layout_passes=False),
      scratch_types=[
          pltpu.VMEM((40, GRP), jnp.int32),
          pltpu.VMEM((GRP,), jnp.float32),
          pltpu.VMEM((slice_w,), jnp.float32),
          pltpu.VMEM_SHARED((NPAD,), jnp.float32),
          pltpu.SemaphoreType.DMA,
      ],
  )
  def k(col_ref, out_ref, idx_v, ones_v, stage_v, hist_s, sem):
    core = lax.axis_index("c")
    sid = lax.axis_index("s")
    wid = sid * NCORES + core
    one = jnp.full((16,), 1.0, jnp.float32)
    zero = jnp.zeros((16,), jnp.float32)
    for i in range(GRP // 16):
      ones_v[pl.ds(i * 16, 16)] = one

    def zbody(j, c):
      stage_v[pl.ds(j * 16, 16)] = zero
      return c
    lax.fori_loop(0, slice_w // 16, zbody, 0)
    pltpu.sync_copy(stage_v, hist_s.at[pl.ds(sid * slice_w, slice_w)])
    plsc.subcore_barrier()

    r0 = wid * rows_per_w

    def body(i, c):
      pltpu.sync_copy(col_ref.at[pl.ds(r0 + i * 40, 40)], idx_v)
      descs = [pltpu.async_copy(ones_v, hist_s.at[idx_v.at[s]], sem, add=True)
               for s in range(40)]
      for d in descs:
        d.wait()
      return c
    lax.fori_loop(0, n_iters, body, 0)
    plsc.subcore_barrier()
    pltpu.sync_copy(hist_s.at[pl.ds(sid * slice_w, slice_w)],
                    out_ref.at[core, pl.ds(sid * slice_w, slice_w)])

  return k(col2d)


def _sc_aggregate(fp):
  nsr = fp // 128
  chunk = {128: 7168, 256: 3584, 384: 2048}[fp]
  nchunk = NPAD // chunk
  ne = GRP // nsr
  pend_n = ((ne + 32 + 15) // 16) * 16
  rpt = chunk * nsr // NSUB

  @functools.partial(
      pl.kernel,
      out_type=jax.ShapeDtypeStruct((NPAD * nsr, 128), jnp.float32),
      mesh=_sc_mesh(),
      compiler_params=pltpu.CompilerParams(needs_layout_passes=False),
      scratch_types=[
          pltpu.VMEM((EBLK,), jnp.int32),
          pltpu.VMEM((EBLK,), jnp.int32),
          pltpu.VMEM((pend_n,), jnp.int32),
          pltpu.VMEM((GRP,), jnp.int32),
          pltpu.VMEM((GRP,), jnp.int32),
          pltpu.VMEM((1, GRP), jnp.int32),
          pltpu.VMEM((1, GRP), jnp.int32),
          pltpu.VMEM((GRP, 128), jnp.float32),
          pltpu.VMEM((GRP, 128), jnp.float32),
          pltpu.VMEM_SHARED((chunk * nsr + 8, 128), jnp.float32),
          pltpu.SemaphoreType.DMA,
      ],
  )
  def k(y_ref, row_ref, col_ref, out_ref,
        colv, rowv, pend, frow0, frow1, fcol0, fcol1, rows0, rows1,
        chunk_s, sem):
    core = lax.axis_index("c")
    sid = lax.axis_index("s")
    ebase = sid * EPT
    iota16 = lax.broadcasted_iota(jnp.int32, (16,), 0)
    frows = (frow0, frow1)
    fcols = (fcol0, fcol1)
    rows = (rows0, rows1)

    def issue(b, lo):
      for i in range(GRP // 16):
        sub = iota16 + i * 16
        e = sub // nsr
        kk = sub - e * nsr
        pv = plsc.load_gather(pend, [e])
        fr = (pv >> 14) * nsr + kk
        fc = (pv & 16383) * nsr + kk
        if GRP % nsr != 0:
          vok = sub < ne * nsr
          fr = jnp.where(vok, fr, sub - 112)
          fc = jnp.where(vok, fc, chunk * nsr + (sub - (ne * nsr)))
        frows[b][pl.ds(i * 16, 16)] = fr
        fcols[b][0, pl.ds(i * 16, 16)] = fc
      pltpu.async_copy(y_ref.at[frows[b]], rows[b], sem)

    def drain(b):
      pltpu.make_async_copy(y_ref.at[frows[b]], rows[b], sem).wait()
      pltpu.sync_copy(rows[b], chunk_s.at[fcols[b].at[0]], add=True)

    def chunk_body(ci, carry):
      cid = ci * NCORES + core
      lo = cid * chunk
      lo_s = cid * chunk * nsr
      pltpu.sync_copy(y_ref.at[pl.ds(lo_s + sid * rpt, rpt)],
                      chunk_s.at[pl.ds(sid * rpt, rpt)])
      plsc.subcore_barrier()

      def vec_body(j, st):
        pos, fcnt = st
        c16 = colv[pl.ds(j * 16, 16)]
        r16 = rowv[pl.ds(j * 16, 16)]
        m = (c16 >= lo) & (c16 < lo + chunk)
        val = r16 * 16384 + (c16 - lo)
        pfx = plsc.cumsum(m.astype(jnp.int32))
        dst = pos + pfx - 1
        plsc.store_scatter(pend, [dst], val, mask=m)
        pos = pos + pfx[15]
        full = pos >= ne

        @pl.when(full)
        def _():
          par = fcnt & 1
          havep = fcnt > 0

          @pl.when(havep & (par == 1))
          def _():
            drain(0)

          @pl.when(havep & (par == 0))
          def _():
            drain(1)

          @pl.when(par == 0)
          def _():
            issue(0, lo)

          @pl.when(par == 1)
          def _():
            issue(1, lo)

          pend[pl.ds(0, 16)] = pend[pl.ds(ne, 16)]

        pos = jnp.where(full, pos - ne, pos)
        fcnt = jnp.where(full, fcnt + 1, fcnt)
        return (pos, fcnt)

      def blk_body(bi, st):
        eb = ebase + bi * EBLK
        pltpu.sync_copy(col_ref.at[pl.ds(eb, EBLK)], colv)
        pltpu.sync_copy(row_ref.at[pl.ds(eb, EBLK)], rowv)
        return lax.fori_loop(0, EBLK // 16, vec_body, st)

      pos, fcnt = lax.fori_loop(0, NBLK, blk_body,
                                (jnp.int32(0), jnp.int32(0)))

      for i in range(-(-ne // 16)):
        off = i * 16
        lanes = iota16 + off
        sel = lanes < pos
        cc = pend[pl.ds(off, 16)]
        pend[pl.ds(off, 16)] = jnp.where(sel, cc, lanes * 16384 + chunk)

      par = fcnt & 1
      havep = fcnt > 0

      @pl.when(havep & (par == 1))
      def _():
        drain(0)

      @pl.when(havep & (par == 0))
      def _():
        drain(1)

      @pl.when(par == 0)
      def _():
        issue(0, lo)
        drain(0)

      @pl.when(par == 1)
      def _():
        issue(1, lo)
        drain(1)

      plsc.subcore_barrier()
      pltpu.sync_copy(chunk_s.at[pl.ds(sid * rpt, rpt)],
                      out_ref.at[pl.ds(lo_s + sid * rpt, rpt)])
      return carry

    lax.fori_loop(0, nchunk // NCORES, chunk_body, 0)

  return k


def _tc_dinv(parts):
  p3 = parts.reshape(2, NPAD // 128, 128)

  def body(p_ref, o_ref):
    deg = 1.0 + p_ref[0] + p_ref[1]
    o_ref[...] = lax.rsqrt(deg)

  out = pl.pallas_call(
      body,
      out_shape=jax.ShapeDtypeStruct((NPAD // 128, 128), jnp.float32),
  )(p3)
  return out.reshape(NPAD, 1)


_MBLK = 512


def _tc_mm_scale(x, w, dinv):
  fi, fo = w.shape

  def body(x_ref, d_ref, w_ref, o_ref):
    o_ref[...] = jnp.dot(x_ref[...], w_ref[...],
                         preferred_element_type=jnp.float32) * d_ref[...]

  return pl.pallas_call(
      body,
      grid=(NPAD // _MBLK,),
      in_specs=[
          pl.BlockSpec((_MBLK, fi), lambda i: (i, 0)),
          pl.BlockSpec((_MBLK, 1), lambda i: (i, 0)),
          pl.BlockSpec((fi, fo), lambda i: (0, 0)),
      ],
      out_specs=pl.BlockSpec((_MBLK, fo), lambda i: (i, 0)),
      out_shape=jax.ShapeDtypeStruct((NPAD, fo), jnp.float32),
  )(x, dinv, w)


def _tc_gcn_mm(agg, dinv, b8, w):
  fi, fo = w.shape

  def body(a_ref, d_ref, b_ref, w_ref, o_ref):
    d = d_ref[...]
    h = jnp.maximum(a_ref[...] * d + b_ref[0:1, :], 0.0)
    o_ref[...] = jnp.dot(h, w_ref[...],
                         preferred_element_type=jnp.float32) * d

  return pl.pallas_call(
      body,
      grid=(NPAD // _MBLK,),
      in_specs=[
          pl.BlockSpec((_MBLK, fi), lambda i: (i, 0)),
          pl.BlockSpec((_MBLK, 1), lambda i: (i, 0)),
          pl.BlockSpec((8, fi), lambda i: (0, 0)),
          pl.BlockSpec((fi, fo), lambda i: (0, 0)),
      ],
      out_specs=pl.BlockSpec((_MBLK, fo), lambda i: (i, 0)),
      out_shape=jax.ShapeDtypeStruct((NPAD, fo), jnp.float32),
  )(agg, dinv, b8, w)


def _tc_segmax(agg, dinv, b8, bv):
  fp = agg.shape[1]

  def body(a_ref, d_ref, b_ref, bv_ref, o_ref):
    i = pl.program_id(0)

    @pl.when(i == 0)
    def _():
      o_ref[...] = jnp.full((BATCH, fp), NEG, jnp.float32)

    h = jnp.maximum(a_ref[...] * d_ref[...] + b_ref[0:1, :], 0.0)
    rows = i * _MBLK + lax.broadcasted_iota(jnp.int32, (_MBLK, 1), 0)
    h = jnp.where(rows < NN, h, NEG)
    bvb = bv_ref[...]
    bmin = bv_ref[0, 0]
    bmax = bv_ref[_MBLK - 1, 0]

    def b_body(bb, c):
      hm = jnp.where(bvb == bb, h, NEG)
      red = jnp.max(hm, axis=0, keepdims=True)
      o_ref[pl.ds(bb, 1), :] = jnp.maximum(o_ref[pl.ds(bb, 1), :], red)
      return c

    lax.fori_loop(bmin, bmax + 1, b_body, 0)

  return pl.pallas_call(
      body,
      grid=(NPAD // _MBLK,),
      in_specs=[
          pl.BlockSpec((_MBLK, fp), lambda i: (i, 0)),
          pl.BlockSpec((_MBLK, 1), lambda i: (i, 0)),
          pl.BlockSpec((8, fp), lambda i: (0, 0)),
          pl.BlockSpec((_MBLK, 1), lambda i: (i, 0)),
      ],
      out_specs=pl.BlockSpec((BATCH, fp), lambda i: (0, 0)),
      out_shape=jax.ShapeDtypeStruct((BATCH, fp), jnp.float32),
  )(agg, dinv, b8, bv)


def _tc_fcg(seg, w1, b1, w2, b2):
  def body(s_ref, w1_ref, b1_ref, w2_ref, b2_ref, o_ref):
    h = jnp.maximum(
        jnp.dot(s_ref[...], w1_ref[...], preferred_element_type=jnp.float32)
        + b1_ref[0:1, :], 0.0)
    o_ref[...] = jnp.dot(h, w2_ref[...],
                         preferred_element_type=jnp.float32) + b2_ref[0:1, :]

  return pl.pallas_call(
      body,
      out_shape=jax.ShapeDtypeStruct((BATCH, w2.shape[1]), jnp.float32),
  )(seg, w1, b1, w2, b2)


_TB = 128


def _tc_conv_halo(xs, wT, b128, k, stride, tout):
  co = wT.shape[1]
  grid = -(-tout // _TB)
  halo = (k - 1) if stride == 1 else (k - 1) // 2
  hl = halo * BATCH
  wblk = _TB * BATCH

  def body(*refs):
    n = len(xs)
    x_refs = refs[:2 * n]
    wT_ref, b_ref, o_ref = refs[2 * n], refs[2 * n + 1], refs[2 * n + 2]
    cats = []
    for t in range(n):
      cur = x_refs[2 * t][...]
      nxt = x_refs[2 * t + 1][:, 0:hl]
      cats.append(jnp.concatenate([cur, nxt], axis=1))
    y = None
    for j in range(k):
      if stride == 1:
        sl = cats[0][:, j * BATCH:j * BATCH + wblk]
      else:
        sl = cats[j % 2][:, (j // 2) * BATCH:(j // 2) * BATCH + wblk]
      t = jnp.dot(wT_ref[j], sl, preferred_element_type=jnp.float32)
      y = t if y is None else y + t
    o_ref[...] = y + b_ref[:, 0:1]

  in_specs = []
  for x in xs:
    ci = x.shape[0]
    in_specs.append(pl.BlockSpec((ci, wblk), lambda i: (0, i)))
    in_specs.append(pl.BlockSpec((ci, wblk), lambda i: (0, i + 1)))
  in_specs.append(pl.BlockSpec(wT.shape, lambda i: (0, 0, 0)))
  in_specs.append(pl.BlockSpec(b128.shape, lambda i: (0, 0)))

  args = []
  for x in xs:
    args += [x, x]
  return pl.pallas_call(
      body,
      grid=(grid,),
      in_specs=in_specs,
      out_specs=pl.BlockSpec((co, wblk), lambda i: (0, i)),
      out_shape=jax.ShapeDtypeStruct((co, tout * BATCH), jnp.float32),
  )(*args, wT, b128)


def _tc_conv_whole(x, wT, b128, k, tout):
  co = wT.shape[1]
  wout = tout * BATCH

  def body(x_ref, wT_ref, b_ref, o_ref):
    xx = x_ref[...]
    y = None
    for j in range(k):
      sl = xx[:, j * BATCH:j * BATCH + wout]
      t = jnp.dot(wT_ref[j], sl, preferred_element_type=jnp.float32)
      y = t if y is None else y + t
    o_ref[...] = y + b_ref[:, 0:1]

  return pl.pallas_call(
      body,
      out_shape=jax.ShapeDtypeStruct((co, wout), jnp.float32),
  )(x, wT, b128)


def _tc_pool(phases):
  co, w = phases[0].shape
  wblk = min(w, 16 * 1024)
  grid = -(-w // wblk)

  def body(*refs):
    o_ref = refs[-1]
    m = refs[0][...]
    for r in refs[1:-1]:
      m = jnp.maximum(m, r[...])
    o_ref[...] = m

  return pl.pallas_call(
      body,
      grid=(grid,),
      in_specs=[pl.BlockSpec((co, wblk), lambda i: (0, i))
                for _ in phases],
      out_specs=pl.BlockSpec((co, wblk), lambda i: (0, i)),
      out_shape=jax.ShapeDtypeStruct((co, w), jnp.float32),
  )(*phases)


_CNN_CFG = [
    (8, 2, 3, 32, 8497, 2832),
    (8, 2, 3, 32, 1413, 471),
    (4, 1, 3, 64, 468, 156),
    (4, 1, 2, 64, 153, 76),
    (4, 1, 2, 128, 73, 36),
    (2, 1, 2, 128, 35, 17),
    (2, 1, 2, 64, 16, 8),
]


def _cnn_branch(target, p):
  x3 = target.T[None]
  for li, (k, s, pool, co, tout, tpool) in enumerate(_CNN_CFG):
    wT = jnp.transpose(p[f"c{li + 1}_W"], (2, 0, 1))
    b128 = jnp.tile(p[f"c{li + 1}_b"][:, None], (1, BATCH))
    ci = x3.shape[0]
    if s == 2:
      grid = -(-tout // _TB)
      tpad = (grid + 1) * _TB
      xe = x3[:, 0::2, :]
      xo = x3[:, 1::2, :]
      xe = jnp.pad(xe, ((0, 0), (0, tpad - xe.shape[1]), (0, 0)))
      xo = jnp.pad(xo, ((0, 0), (0, tpad - xo.shape[1]), (0, 0)))
      y = _tc_conv_halo([xe.reshape(ci, -1), xo.reshape(ci, -1)],
                        wT, b128, k, 2, tout)
    else:
      y = _tc_conv_whole(x3.reshape(ci, -1), wT, b128, k, tout)
    y3 = y.reshape(co, tout, BATCH)
    phases = [y3[:, j:tpool * pool:pool, :].reshape(co, -1)
              for j in range(pool)]
    yp = _tc_pool(phases)
    x3 = yp.reshape(co, tpool, BATCH)
  return x3.reshape(512, BATCH).T


def _tc_head(g1, g2, flatT, wx, bx, w1a, w1b, w1c, b1, w2, b2, w3, b3):
  def body(g1_ref, g2_ref, f_ref, wx_ref, bx_ref, w1a_ref, w1b_ref, w1c_ref,
           b1_ref, w2_ref, b2_ref, w3_ref, b3_ref, o_ref):
    f32 = jnp.float32
    xt = jnp.dot(f_ref[...], wx_ref[...],
                 preferred_element_type=f32) + bx_ref[0:1, :]
    h = (jnp.dot(g1_ref[...], w1a_ref[...], preferred_element_type=f32)
         + jnp.dot(g2_ref[...], w1b_ref[...], preferred_element_type=f32)
         + jnp.dot(xt, w1c_ref[...], preferred_element_type=f32)
         + b1_ref[0:1, :])
    h = jnp.maximum(h, 0.0)
    h = jnp.maximum(
        jnp.dot(h, w2_ref[...], preferred_element_type=f32) + b2_ref[0:1, :],
        0.0)
    o_ref[...] = jnp.dot(h, w3_ref[...],
                         preferred_element_type=f32) + b3_ref[0:1, :]

  return pl.pallas_call(
      body,
      out_shape=jax.ShapeDtypeStruct((BATCH, 1), jnp.float32),
  )(g1, g2, flatT, wx, bx, w1a, w1b, w1c, b1, w2, b2, w3, b3)


def _pad_rows_cols(w, fi, fo):
  return jnp.pad(w, ((0, fi - w.shape[0]), (0, fo - w.shape[1])))


def _bias8(b, fo):
  return jnp.tile(jnp.pad(b, (0, fo - b.shape[0]))[None, :], (8, 1))


def _branch(x, ei, bv, p, fps):
  f1, f2, f3 = fps
  row = ei[0]
  col = ei[1]
  col_pad = jnp.concatenate(
      [col, NN + (jnp.arange(DEG_ROWS * GRP - EE, dtype=jnp.int32) % 7000)])
  parts = _sc_degree(col_pad.reshape(DEG_ROWS, GRP))
  dinv = _tc_dinv(parts)

  def agg_call(y, fp):
    y2 = y.reshape(NPAD * (fp // 128), 128)
    return _sc_aggregate(fp)(y2, row, col).reshape(NPAD, fp)

  x_pad = jnp.pad(x, ((0, NPAD - NN), (0, f1 - x.shape[1])))
  w1 = _pad_rows_cols(p["gcn1_W"], f1, f1)
  y1 = _tc_mm_scale(x_pad, w1, dinv)
  agg1 = agg_call(y1, f1)

  w2 = _pad_rows_cols(p["gcn2_W"], f1, f2)
  y2 = _tc_gcn_mm(agg1, dinv, _bias8(p["gcn1_b"], f1), w2)
  agg2 = agg_call(y2, f2)

  w3 = _pad_rows_cols(p["gcn3_W"], f2, f3)
  y3 = _tc_gcn_mm(agg2, dinv, _bias8(p["gcn2_b"], f2), w3)
  agg3 = agg_call(y3, f3)

  bv_pad = jnp.pad(bv, (0, NPAD - NN), constant_values=BATCH - 1)
  seg = _tc_segmax(agg3, dinv, _bias8(p["gcn3_b"], f3), bv_pad[:, None])

  w_f1 = jnp.pad(p["fcg1_W"], ((0, f3 - p["fcg1_W"].shape[0]), (0, 0)))
  return _tc_fcg(seg, w_f1, _bias8(p["fcg1_b"], 1024),
                 p["fcg2_W"], _bias8(p["fcg2_b"], 128))


def kernel(x_1, edge_index_1, x_1_batch, x_2, edge_index_2, x_2_batch,
           target, params):
  p = params
  fps = (128, 256, 384)
  g1 = _branch(x_1, edge_index_1, x_1_batch, p, fps)
  g2 = _branch(x_2, edge_index_2, x_2_batch, p, fps)

  flatT = _cnn_branch(target, p)

  w1 = p["fc1_W"]
  out = _tc_head(g1, g2, flatT, p["fcxt_W"], _bias8(p["fcxt_b"], 128),
                 w1[0:128], w1[128:256], w1[256:384],
                 _bias8(p["fc1_b"], 1024), p["fc2_W"],
                 _bias8(p["fc2_b"], 128), p["out_W"], _bias8(p["out_b"], 1))
  return out
